# baseline (speedup 1.0000x reference)
import functools

import jax
import jax.numpy as jnp
from jax.experimental import pallas as pl
from jax.experimental.pallas import tpu as pltpu

D_MODEL = 4096
N_EXP = 64
K = 8
T_BLK = 1024
D_BLK = 1024
N_D = D_MODEL // D_BLK


def _router_body(x_ref, wt_ref, idx_ref, gate_ref, logits_ref):
    k = pl.program_id(1)
    part = jnp.dot(x_ref[...], wt_ref[...], preferred_element_type=jnp.float32)

    @pl.when(k == 0)
    def _():
        logits_ref[...] = part

    @pl.when(k > 0)
    def _():
        logits_ref[...] += part

    @pl.when(k == N_D - 1)
    def _():
        logits = logits_ref[...]
        lt = logits.T
        iota = jax.lax.broadcasted_iota(jnp.int32, lt.shape, 0).astype(jnp.float32)
        cur = lt
        vals = []
        idxs = []
        for _ in range(K):
            m = jnp.max(cur, axis=0, keepdims=True)
            amax = jnp.min(
                jnp.where(cur == m, iota, jnp.float32(N_EXP)), axis=0, keepdims=True
            )
            vals.append(m)
            idxs.append(amax)
            cur = jnp.where(iota == amax, -jnp.inf, cur)

        tv = jnp.concatenate(vals, axis=0)
        ti = jnp.concatenate(idxs, axis=0)
        ev = jnp.exp(tv - tv[0:1, :])
        g = ev / jnp.sum(ev, axis=0, keepdims=True)
        gate_ref[...] = g.T
        idx_ref[...] = ti.T.astype(jnp.int32)


@jax.jit
def kernel(x, router_weights):
    b, s, d = x.shape
    n_tok = b * s
    x2 = x.reshape(n_tok, d)
    wt = router_weights.T

    grid = (n_tok // T_BLK, N_D)
    idx_out, gates, logits = pl.pallas_call(
        _router_body,
        grid=grid,
        in_specs=[
            pl.BlockSpec((T_BLK, D_BLK), lambda i, k: (i, k)),
            pl.BlockSpec((D_BLK, N_EXP), lambda i, k: (k, 0)),
        ],
        out_specs=[
            pl.BlockSpec((T_BLK, K), lambda i, k: (i, 0)),
            pl.BlockSpec((T_BLK, K), lambda i, k: (i, 0)),
            pl.BlockSpec((T_BLK, N_EXP), lambda i, k: (i, 0)),
        ],
        out_shape=[
            jax.ShapeDtypeStruct((n_tok, K), jnp.int32),
            jax.ShapeDtypeStruct((n_tok, K), jnp.float32),
            jax.ShapeDtypeStruct((n_tok, N_EXP), jnp.float32),
        ],
        compiler_params=pltpu.CompilerParams(
            dimension_semantics=("parallel", "arbitrary"),
        ),
    )(x2, wt)

    return (
        idx_out.reshape(b, s, K),
        gates.reshape(b, s, K),
        logits.reshape(b, s, N_EXP),
    )


# speedup vs baseline: 1.3163x; 1.3163x over previous
import functools

import jax
import jax.numpy as jnp
from jax.experimental import pallas as pl

D_MODEL = 4096
N_EXP = 64
K = 8
T_BLK = 1024


def _router_body(x_ref, wt_ref, idx_ref, gate_ref, logits_ref):
    logits = jax.lax.dot_general(
        x_ref[...], wt_ref[...], (((1,), (1,)), ((), ())),
        preferred_element_type=jnp.float32)
    logits_ref[...] = logits

    lt = logits.T
    iota = jax.lax.broadcasted_iota(jnp.int32, lt.shape, 0).astype(jnp.float32)
    cur = lt
    vals = []
    idxs = []
    for _ in range(K):
        m = jnp.max(cur, axis=0, keepdims=True)
        amax = jnp.min(
            jnp.where(cur == m, iota, jnp.float32(N_EXP)), axis=0, keepdims=True
        )
        vals.append(m)
        idxs.append(amax)
        cur = jnp.where(iota == amax, -jnp.inf, cur)

    tv = jnp.concatenate(vals, axis=0)
    ti = jnp.concatenate(idxs, axis=0)
    ev = jnp.exp(tv - tv[0:1, :])
    g = ev / jnp.sum(ev, axis=0, keepdims=True)
    gate_ref[...] = g.T
    idx_ref[...] = ti.T.astype(jnp.int32)


@jax.jit
def kernel(x, router_weights):
    b, s, d = x.shape
    n_tok = b * s
    x2 = x.reshape(n_tok, d)
    wt = router_weights

    grid = (n_tok // T_BLK,)
    idx_out, gates, logits = pl.pallas_call(
        _router_body,
        grid=grid,
        in_specs=[
            pl.BlockSpec((T_BLK, d), lambda i: (i, 0)),
            pl.BlockSpec((N_EXP, d), lambda i: (0, 0)),
        ],
        out_specs=[
            pl.BlockSpec((T_BLK, K), lambda i: (i, 0)),
            pl.BlockSpec((T_BLK, K), lambda i: (i, 0)),
            pl.BlockSpec((T_BLK, N_EXP), lambda i: (i, 0)),
        ],
        out_shape=[
            jax.ShapeDtypeStruct((n_tok, K), jnp.int32),
            jax.ShapeDtypeStruct((n_tok, K), jnp.float32),
            jax.ShapeDtypeStruct((n_tok, N_EXP), jnp.float32),
        ],
    )(x2, wt)

    return (
        idx_out.reshape(b, s, K),
        gates.reshape(b, s, K),
        logits.reshape(b, s, N_EXP),
    )
